# baseline (device time: 177333 ns/iter reference)
import jax
import jax.numpy as jnp
from jax import lax
from jax.experimental import pallas as pl
from jax.experimental.pallas import tpu as pltpu

N_DEV = 4
N_SLOTS = 3
BQ = 512


def kernel(q, k, v):
    s_per, d = q.shape
    scale = 1.0 / (d ** 0.5)
    n_blocks = s_per // BQ

    def body(q_ref, k_ref, v_ref, out_ref, qs_ref, comm_ref, l_ref,
             send_sems, recv_sems, credit_sem):
        my_pos = lax.axis_index("i")
        left = (my_pos - 1) % N_DEV
        right = (my_pos + 1) % N_DEV

        barrier_sem = pltpu.get_barrier_semaphore()
        for nbr in [left, right]:
            pl.semaphore_signal(
                barrier_sem, inc=1,
                device_id=(nbr,), device_id_type=pl.DeviceIdType.MESH,
            )
        pl.semaphore_wait(barrier_sem, 2)

        qs_ref[:, :] = (q_ref[:, :] * scale).astype(jnp.bfloat16)
        comm_ref[0, 0, :, :] = k_ref[:, :].astype(jnp.bfloat16)
        comm_ref[0, 1, :, :] = v_ref[:, :].astype(jnp.bfloat16)
        out_ref[:, :] = jnp.zeros((s_per, d), jnp.float32)
        l_ref[:, :] = jnp.zeros((s_per, 128), jnp.float32)

        def step_compute(slot):
            k_c = comm_ref[slot, 0, :, :]
            v_c = comm_ref[slot, 1, :, :]

            def blk(b, _):
                rows = pl.ds(b * BQ, BQ)
                s = lax.dot_general(
                    qs_ref[rows, :], k_c, (((1,), (1,)), ((), ())),
                    preferred_element_type=jnp.float32,
                )
                p = jnp.exp(s.astype(jnp.bfloat16))
                l_ref[rows, :] = l_ref[rows, :] + jnp.broadcast_to(
                    jnp.sum(p, axis=1, keepdims=True, dtype=jnp.float32),
                    (BQ, 128))
                out_ref[rows, :] = out_ref[rows, :] + jnp.dot(
                    p, v_c, preferred_element_type=jnp.float32)
                return 0

            lax.fori_loop(0, n_blocks, blk, 0)

        for h in range(N_DEV):
            if h < N_DEV - 1:
                if h == N_DEV - 2:
                    pl.semaphore_wait(credit_sem, 1)
                rdma = pltpu.make_async_remote_copy(
                    src_ref=comm_ref.at[h % N_SLOTS],
                    dst_ref=comm_ref.at[(h + 1) % N_SLOTS],
                    send_sem=send_sems.at[h],
                    recv_sem=recv_sems.at[h],
                    device_id=(right,),
                    device_id_type=pl.DeviceIdType.MESH,
                )
                rdma.start()
                step_compute(h % N_SLOTS)
                rdma.wait()
                if h == 0:
                    pl.semaphore_signal(
                        credit_sem, inc=1,
                        device_id=(left,),
                        device_id_type=pl.DeviceIdType.MESH,
                    )
            else:
                step_compute(h % N_SLOTS)

        out_ref[:, :] = out_ref[:, :] / l_ref[:, 0:1]

    return pl.pallas_call(
        body,
        out_shape=jax.ShapeDtypeStruct((s_per, d), jnp.float32),
        in_specs=[pl.BlockSpec(memory_space=pltpu.VMEM)] * 3,
        out_specs=pl.BlockSpec(memory_space=pltpu.VMEM),
        scratch_shapes=[
            pltpu.VMEM((s_per, d), jnp.bfloat16),
            pltpu.VMEM((N_SLOTS, 2, s_per, d), jnp.bfloat16),
            pltpu.VMEM((s_per, 128), jnp.float32),
            pltpu.SemaphoreType.DMA((N_DEV - 1,)),
            pltpu.SemaphoreType.DMA((N_DEV - 1,)),
            pltpu.SemaphoreType.REGULAR,
        ],
        compiler_params=pltpu.CompilerParams(
            collective_id=0,
            vmem_limit_bytes=100 * 1024 * 1024,
        ),
    )(q, k, v)


# device time: 109696 ns/iter; 1.6166x vs baseline; 1.6166x over previous
import jax
import jax.numpy as jnp
from jax import lax
from jax.experimental import pallas as pl
from jax.experimental.pallas import tpu as pltpu

N_DEV = 4
N_SLOTS = 3
BQ = 512


def kernel(q, k, v):
    s_per, d = q.shape
    scale = 1.0 / (d ** 0.5)
    n_blocks = s_per // BQ
    half = s_per // 2

    def body(q_ref, k_ref, v_ref, out_ref, qs_ref, comm_ref, l_ref,
             send_sems, recv_sems, credit_sem):
        my_pos = lax.axis_index("i")
        left = (my_pos - 1) % N_DEV
        right = (my_pos + 1) % N_DEV

        barrier_sem = pltpu.get_barrier_semaphore()
        for nbr in [left, right]:
            pl.semaphore_signal(
                barrier_sem, inc=1,
                device_id=(nbr,), device_id_type=pl.DeviceIdType.MESH,
            )
        pl.semaphore_wait(barrier_sem, 2)

        qs_ref[:, :] = (q_ref[:, :] * scale).astype(jnp.bfloat16)
        comm_ref[0, 0, :, :] = k_ref[:, :].astype(jnp.bfloat16)
        comm_ref[0, 1, :, :] = v_ref[:, :].astype(jnp.bfloat16)
        out_ref[:, :] = jnp.zeros((s_per, d), jnp.float32)
        l_ref[:, :] = jnp.zeros((s_per, 128), jnp.float32)

        def step_compute(slot):
            k_c = comm_ref[slot, 0, :, :]
            v_c = comm_ref[slot, 1, :, :]

            def blk(b, _):
                rows = pl.ds(b * BQ, BQ)
                s = lax.dot_general(
                    qs_ref[rows, :], k_c, (((1,), (1,)), ((), ())),
                    preferred_element_type=jnp.float32,
                )
                p = jnp.exp(s.astype(jnp.bfloat16))
                l_ref[rows, :] = l_ref[rows, :] + jnp.broadcast_to(
                    jnp.sum(p, axis=1, keepdims=True, dtype=jnp.float32),
                    (BQ, 128))
                out_ref[rows, :] = out_ref[rows, :] + jnp.dot(
                    p, v_c, preferred_element_type=jnp.float32)
                return 0

            lax.fori_loop(0, n_blocks, blk, 0)

        def make_hop(h):
            src, dst = h % N_SLOTS, (h + 1) % N_SLOTS
            cw = pltpu.make_async_remote_copy(
                src_ref=comm_ref.at[src, :, pl.ds(0, half)],
                dst_ref=comm_ref.at[dst, :, pl.ds(0, half)],
                send_sem=send_sems.at[h, 0],
                recv_sem=recv_sems.at[h, 0],
                device_id=(right,),
                device_id_type=pl.DeviceIdType.MESH,
            )
            ccw = pltpu.make_async_remote_copy(
                src_ref=comm_ref.at[src, :, pl.ds(half, half)],
                dst_ref=comm_ref.at[dst, :, pl.ds(half, half)],
                send_sem=send_sems.at[h, 1],
                recv_sem=recv_sems.at[h, 1],
                device_id=(left,),
                device_id_type=pl.DeviceIdType.MESH,
            )
            return cw, ccw

        for h in range(N_DEV):
            if h < N_DEV - 1:
                if h == N_DEV - 2:
                    pl.semaphore_wait(credit_sem, 2)
                cw, ccw = make_hop(h)
                cw.start()
                ccw.start()
                step_compute(h % N_SLOTS)
                cw.wait()
                ccw.wait()
                if h == 0:
                    for nbr in [left, right]:
                        pl.semaphore_signal(
                            credit_sem, inc=1,
                            device_id=(nbr,),
                            device_id_type=pl.DeviceIdType.MESH,
                        )
            else:
                step_compute(h % N_SLOTS)

        out_ref[:, :] = out_ref[:, :] / l_ref[:, 0:1]

    return pl.pallas_call(
        body,
        out_shape=jax.ShapeDtypeStruct((s_per, d), jnp.float32),
        in_specs=[pl.BlockSpec(memory_space=pltpu.VMEM)] * 3,
        out_specs=pl.BlockSpec(memory_space=pltpu.VMEM),
        scratch_shapes=[
            pltpu.VMEM((s_per, d), jnp.bfloat16),
            pltpu.VMEM((N_SLOTS, 2, s_per, d), jnp.bfloat16),
            pltpu.VMEM((s_per, 128), jnp.float32),
            pltpu.SemaphoreType.DMA((N_DEV - 1, 2)),
            pltpu.SemaphoreType.DMA((N_DEV - 1, 2)),
            pltpu.SemaphoreType.REGULAR,
        ],
        compiler_params=pltpu.CompilerParams(
            collective_id=0,
            vmem_limit_bytes=100 * 1024 * 1024,
        ),
    )(q, k, v)


# device time: 107121 ns/iter; 1.6554x vs baseline; 1.0240x over previous
import jax
import jax.numpy as jnp
from jax import lax
from jax.experimental import pallas as pl
from jax.experimental.pallas import tpu as pltpu

N_DEV = 4
N_SLOTS = 3
BQ = 512


def kernel(q, k, v):
    s_per, d = q.shape
    scale = 1.0 / (d ** 0.5)
    n_blocks = s_per // BQ
    half = s_per // 2

    def body(q_ref, k_ref, v_ref, out_ref, qs_ref, comm_ref, l_ref,
             send_sems, recv_sems, credit_sem):
        my_pos = lax.axis_index("i")
        left = (my_pos - 1) % N_DEV
        right = (my_pos + 1) % N_DEV

        barrier_sem = pltpu.get_barrier_semaphore()
        for nbr in [left, right]:
            pl.semaphore_signal(
                barrier_sem, inc=1,
                device_id=(nbr,), device_id_type=pl.DeviceIdType.MESH,
            )
        pl.semaphore_wait(barrier_sem, 2)

        qs_ref[:, :] = (q_ref[:, :] * scale).astype(jnp.bfloat16)
        comm_ref[0, 0, :, :] = k_ref[:, :].astype(jnp.bfloat16)
        comm_ref[0, 1, :, :] = v_ref[:, :].astype(jnp.bfloat16)
        out_ref[:, :] = jnp.zeros((s_per, d), jnp.float32)
        l_ref[:, :] = jnp.zeros((s_per, 128), jnp.float32)

        def step_compute(slot):
            k_c = comm_ref[slot, 0, :, :]
            v_c = comm_ref[slot, 1, :, :]

            for b in range(n_blocks):
                rows = pl.ds(b * BQ, BQ)
                s = lax.dot_general(
                    qs_ref[rows, :], k_c, (((1,), (1,)), ((), ())),
                    preferred_element_type=jnp.float32,
                )
                p = jnp.exp(s.astype(jnp.bfloat16))
                l_ref[rows, :] = l_ref[rows, :] + jnp.broadcast_to(
                    jnp.sum(p, axis=1, keepdims=True, dtype=jnp.float32),
                    (BQ, 128))
                out_ref[rows, :] = out_ref[rows, :] + jnp.dot(
                    p, v_c, preferred_element_type=jnp.float32)

        def make_hop(h):
            src, dst = h % N_SLOTS, (h + 1) % N_SLOTS
            cw = pltpu.make_async_remote_copy(
                src_ref=comm_ref.at[src, :, pl.ds(0, half)],
                dst_ref=comm_ref.at[dst, :, pl.ds(0, half)],
                send_sem=send_sems.at[h, 0],
                recv_sem=recv_sems.at[h, 0],
                device_id=(right,),
                device_id_type=pl.DeviceIdType.MESH,
            )
            ccw = pltpu.make_async_remote_copy(
                src_ref=comm_ref.at[src, :, pl.ds(half, half)],
                dst_ref=comm_ref.at[dst, :, pl.ds(half, half)],
                send_sem=send_sems.at[h, 1],
                recv_sem=recv_sems.at[h, 1],
                device_id=(left,),
                device_id_type=pl.DeviceIdType.MESH,
            )
            return cw, ccw

        for h in range(N_DEV):
            if h < N_DEV - 1:
                if h == N_DEV - 2:
                    pl.semaphore_wait(credit_sem, 2)
                cw, ccw = make_hop(h)
                cw.start()
                ccw.start()
                step_compute(h % N_SLOTS)
                cw.wait()
                ccw.wait()
                if h == 0:
                    for nbr in [left, right]:
                        pl.semaphore_signal(
                            credit_sem, inc=1,
                            device_id=(nbr,),
                            device_id_type=pl.DeviceIdType.MESH,
                        )
            else:
                step_compute(h % N_SLOTS)

        out_ref[:, :] = out_ref[:, :] / l_ref[:, 0:1]

    return pl.pallas_call(
        body,
        out_shape=jax.ShapeDtypeStruct((s_per, d), jnp.float32),
        in_specs=[pl.BlockSpec(memory_space=pltpu.VMEM)] * 3,
        out_specs=pl.BlockSpec(memory_space=pltpu.VMEM),
        scratch_shapes=[
            pltpu.VMEM((s_per, d), jnp.bfloat16),
            pltpu.VMEM((N_SLOTS, 2, s_per, d), jnp.bfloat16),
            pltpu.VMEM((s_per, 128), jnp.float32),
            pltpu.SemaphoreType.DMA((N_DEV - 1, 2)),
            pltpu.SemaphoreType.DMA((N_DEV - 1, 2)),
            pltpu.SemaphoreType.REGULAR,
        ],
        compiler_params=pltpu.CompilerParams(
            collective_id=0,
            vmem_limit_bytes=100 * 1024 * 1024,
        ),
    )(q, k, v)
